# Initial kernel scaffold; baseline (speedup 1.0000x reference)
import jax
import jax.numpy as jnp
from jax.experimental import pallas as pl


def kernel(x, is_visual, W, aux_free_bias, modality_bias):
    raise NotImplementedError("write your pallas kernel here")


# speedup vs baseline: 1.6457x; 1.6457x over previous
import functools

import jax
import jax.numpy as jnp
from jax.experimental import pallas as pl
from jax.experimental.pallas import tpu as pltpu

_TB = 512


def _router_block(x_ref, visf_ref, w_ref, aux_ref, mb_ref,
                  idx_ref, gate_ref, aff_ref, *, n_experts, top_k):
    x = x_ref[...]
    logits = jnp.dot(x, w_ref[...], preferred_element_type=jnp.float32)
    sp = jnp.maximum(logits, 0.0) + jnp.log1p(jnp.exp(-jnp.abs(logits)))
    aff = jnp.sqrt(sp + 1e-12)

    visf = visf_ref[...]
    mb0 = mb_ref[0:1, :]
    mb1 = mb_ref[1:2, :]
    mrow = jnp.where(visf > 0.5, mb1, mb0)
    biased = aff + aux_ref[...] + mrow

    tb = biased.shape[0]
    iota = jax.lax.broadcasted_iota(jnp.int32, (tb, n_experts), 1)
    neg_inf = jnp.float32(-jnp.inf)

    idx_cols = []
    gate_cols = []
    work = biased
    for _ in range(top_k):
        m = jnp.max(work, axis=-1, keepdims=True)
        cand = jnp.where(work == m, iota, n_experts)
        sel = jnp.min(cand, axis=-1, keepdims=True)
        pick = iota == sel
        g = jnp.sum(jnp.where(pick, aff, 0.0), axis=-1, keepdims=True)
        idx_cols.append(sel)
        gate_cols.append(g)
        work = jnp.where(pick, neg_inf, work)

    idx = jnp.concatenate(idx_cols, axis=1)
    gate_raw = jnp.concatenate(gate_cols, axis=1)
    gate = gate_raw / (jnp.sum(gate_raw, axis=-1, keepdims=True) + 1e-12)

    idx_ref[...] = idx
    gate_ref[...] = gate
    aff_ref[...] = aff


def kernel(x, is_visual, W, aux_free_bias, modality_bias):
    T, D = x.shape
    E = W.shape[1]
    top_k = 8
    tb = _TB
    grid = (T // tb,)

    visf = is_visual.astype(jnp.float32).reshape(T, 1)
    aux2 = aux_free_bias.reshape(1, E)

    body = functools.partial(_router_block, n_experts=E, top_k=top_k)
    idx, gate, aff = pl.pallas_call(
        body,
        grid=grid,
        in_specs=[
            pl.BlockSpec((tb, D), lambda i: (i, 0)),
            pl.BlockSpec((tb, 1), lambda i: (i, 0)),
            pl.BlockSpec((D, E), lambda i: (0, 0)),
            pl.BlockSpec((1, E), lambda i: (0, 0)),
            pl.BlockSpec((2, E), lambda i: (0, 0)),
        ],
        out_specs=[
            pl.BlockSpec((tb, top_k), lambda i: (i, 0)),
            pl.BlockSpec((tb, top_k), lambda i: (i, 0)),
            pl.BlockSpec((tb, E), lambda i: (i, 0)),
        ],
        out_shape=[
            jax.ShapeDtypeStruct((T, top_k), jnp.int32),
            jax.ShapeDtypeStruct((T, top_k), jnp.float32),
            jax.ShapeDtypeStruct((T, E), jnp.float32),
        ],
    )(x, visf, W, aux2, modality_bias)
    return (idx, gate, aff)
